# baseline (speedup 1.0000x reference)
import functools

import jax
import jax.numpy as jnp
from jax import lax
from jax.experimental import pallas as pl
from jax.experimental.pallas import tpu as pltpu
from jax.experimental.pallas import tpu_sc as plsc

B = 4096
D_IN = 2048
H = 4096
N = 2048
K_UNIVERSE = 512
K_TRADE = 128

BM = 1024
BK = 512
NI = B // BM
NK = H // BK

BS = 512


def _mlp_body(x_ref, w1_ref, b1_ref, w2_ref, b2_ref, out_ref):
    k = pl.program_id(1)

    h = jnp.dot(x_ref[...], w1_ref[...], preferred_element_type=jnp.float32)
    h = jnp.maximum(h + b1_ref[...], 0.0)
    contrib = jnp.dot(h, w2_ref[...], preferred_element_type=jnp.float32)

    @pl.when(k == 0)
    def _init():
        out_ref[...] = contrib

    @pl.when(k > 0)
    def _accum():
        out_ref[...] += contrib

    @pl.when(k == NK - 1)
    def _finish():
        out_ref[...] = jax.nn.sigmoid(out_ref[...] + b2_ref[...])


def _kth_largest_bits(bits, k, nbits):
    def body(j, t):
        cand = t | (jnp.int32(1) << (jnp.int32(nbits - 1) - j))
        cnt = jnp.sum((bits >= cand).astype(jnp.int32), axis=1, keepdims=True)
        return jnp.where(cnt >= k, cand, t)

    t0 = jnp.zeros((bits.shape[0], 1), jnp.int32)
    return jax.lax.fori_loop(0, nbits, body, t0)


SC_NC = 2
SC_NS = 16
SC_NW = SC_NC * SC_NS
SC_RPW = B // SC_NW
SC_RCHUNK = 16


def _rbits_sc_body(vol_hbm, spr_hbm, out_hbm, vol_v, spr_v, rb_v):
    wid = lax.axis_index("s") * SC_NC + lax.axis_index("c")
    base = wid * SC_RPW

    for chunk in range(SC_RPW // SC_RCHUNK):
        r0 = base + chunk * SC_RCHUNK
        pltpu.sync_copy(vol_hbm.at[pl.ds(r0, SC_RCHUNK)], vol_v)
        pltpu.sync_copy(spr_hbm.at[pl.ds(r0, SC_RCHUNK)], spr_v)

        def col_body(j, carry):
            o = j * 16
            for r in range(SC_RCHUNK):
                ratio = vol_v[r, pl.ds(o, 16)] / (spr_v[r, pl.ds(o, 16)]
                                                  + 1e-8)
                rb_v[r, pl.ds(o, 16)] = jax.lax.bitcast_convert_type(
                    ratio, jnp.int32)
            return carry

        jax.lax.fori_loop(0, N // 16, col_body, 0)
        pltpu.sync_copy(rb_v, out_hbm.at[pl.ds(r0, SC_RCHUNK)])


def _rbits_sc(volatility, spread):
    return pl.kernel(
        _rbits_sc_body,
        mesh=plsc.VectorSubcoreMesh(core_axis_name="c",
                                    subcore_axis_name="s"),
        out_type=jax.ShapeDtypeStruct((B, N), jnp.int32),
        scratch_types=[
            pltpu.VMEM((SC_RCHUNK, N), jnp.float32),
            pltpu.VMEM((SC_RCHUNK, N), jnp.float32),
            pltpu.VMEM((SC_RCHUNK, N), jnp.int32),
        ],
    )(volatility, spread)


def _select_body(repr_ref, rbits_ref, out_ref):
    ls = repr_ref[...] - 0.5

    rbits = rbits_ref[...]
    t1 = _kth_largest_bits(rbits, K_UNIVERSE, 31)

    abits = jax.lax.bitcast_convert_type(jnp.abs(ls), jnp.int32)
    cbits = jnp.where(rbits >= t1, abits, jnp.int32(-1))
    t2 = _kth_largest_bits(cbits, K_TRADE, 30)

    sel = jnp.where(cbits >= t2, ls, 0.0)
    denom = jnp.sum(jnp.abs(sel), axis=1, keepdims=True) + 1e-8
    out_ref[...] = sel / denom


@functools.partial(jax.jit, static_argnames=("interpret",))
def _run(signal_features, volatility, spread, W1, b1, W2, b2,
         interpret=False):
    rbits = _rbits_sc(volatility, spread)

    signal_repr = pl.pallas_call(
        _mlp_body,
        grid=(NI, NK),
        in_specs=[
            pl.BlockSpec((BM, D_IN), lambda i, k: (i, 0)),
            pl.BlockSpec((D_IN, BK), lambda i, k: (0, k)),
            pl.BlockSpec((1, BK), lambda i, k: (0, k)),
            pl.BlockSpec((BK, N), lambda i, k: (k, 0)),
            pl.BlockSpec((1, N), lambda i, k: (0, 0)),
        ],
        out_specs=pl.BlockSpec((BM, N), lambda i, k: (i, 0)),
        out_shape=jax.ShapeDtypeStruct((B, N), jnp.float32),
        compiler_params=pltpu.CompilerParams(
            dimension_semantics=("parallel", "arbitrary"),
        ),
        interpret=interpret,
    )(signal_features, W1, b1.reshape(1, H), W2, b2.reshape(1, N))

    action = pl.pallas_call(
        _select_body,
        grid=(B // BS,),
        in_specs=[
            pl.BlockSpec((BS, N), lambda i: (i, 0)),
            pl.BlockSpec((BS, N), lambda i: (i, 0)),
        ],
        out_specs=pl.BlockSpec((BS, N), lambda i: (i, 0)),
        out_shape=jax.ShapeDtypeStruct((B, N), jnp.float32),
        compiler_params=pltpu.CompilerParams(
            dimension_semantics=("parallel",),
        ),
        interpret=interpret,
    )(signal_repr, rbits)
    return action, jnp.zeros_like(action)


def kernel(signal_features, volatility, spread, W1, b1, W2, b2):
    return _run(signal_features, volatility, spread, W1, b1, W2, b2)
